# baseline (speedup 1.0000x reference)
import jax
import jax.numpy as jnp
from jax.experimental import pallas as pl
from jax.experimental.pallas import tpu as pltpu
from jax.experimental.pallas import tpu_sc as plsc

D = 4096
NSHOT = 32
NTOK = 512
KSHOT = 4
KTOK = 1024
FLAT = KSHOT * NTOK

NBLK = 512
MBLK = 1024
SBLK = 256


def _ln(x, scale, bias, eps=1e-5):
    m = jnp.mean(x, axis=-1, keepdims=True)
    v = jnp.var(x, axis=-1, keepdims=True)
    return (x - m) / jnp.sqrt(v + eps) * scale + bias


def _gelu_exact(x):
    return 0.5 * x * (1.0 + jax.lax.erf(x * 0.7071067811865476))


def _softmax1(s):
    m = jnp.max(s, axis=-1, keepdims=True)
    e = jnp.exp(s - m)
    return e[:, 1:2] / (e[:, 0:1] + e[:, 1:2])


def _mean_body(x_ref, o_ref):
    o_ref[...] = jnp.mean(x_ref[...], axis=1)[:, None, :]


def _shot_means(input_embeds):
    out = pl.pallas_call(
        _mean_body,
        grid=(NSHOT,),
        in_specs=[pl.BlockSpec((1, NTOK, D), lambda i: (i, 0, 0))],
        out_specs=pl.BlockSpec((1, 1, D), lambda i: (i, 0, 0)),
        out_shape=jax.ShapeDtypeStruct((NSHOT, 1, D), jnp.float32),
    )(input_embeds)
    return out.reshape(NSHOT, D)


def _topk_positions(p_col, p_row, k, npad, expand=1):
    n = p_col.shape[0]
    c = min(SBLK, n)
    jrow = jax.lax.broadcasted_iota(jnp.int32, (c, n), 1)
    ranks = []
    for b in range(n // c):
        pc = p_col[b * c:(b + 1) * c, :]
        icol = jax.lax.broadcasted_iota(jnp.int32, (c, 1), 0) + b * c
        cond = (p_row > pc) | ((p_row == pc) & (jrow < icol))
        ranks.append(jnp.sum(cond.astype(jnp.float32), axis=1, keepdims=True))
    rank = jnp.concatenate(ranks, axis=0) if len(ranks) > 1 else ranks[0]
    sel = (rank < float(k)).astype(jnp.float32)
    sel_row = jnp.swapaxes(sel, 0, 1)
    cums = []
    for b in range(n // c):
        irow = jax.lax.broadcasted_iota(jnp.int32, (c, 1), 0) + b * c
        tri = (jrow <= irow).astype(jnp.float32)
        cums.append(jnp.sum(tri * sel_row, axis=1, keepdims=True))
    cums = jnp.concatenate(cums, axis=0) if len(cums) > 1 else cums[0]
    krow = jax.lax.broadcasted_iota(jnp.int32, (c, npad), 1)
    acc = jnp.zeros((1, npad), jnp.int32)
    for b in range(n // c):
        cc = cums[b * c:(b + 1) * c, :].astype(jnp.int32)
        acc = acc + jnp.sum((cc <= krow).astype(jnp.int32), axis=0,
                            keepdims=True)
    if expand == 1:
        return acc
    shift = expand.bit_length() - 1
    ke = jax.lax.broadcasted_iota(jnp.int32, (c, npad * expand), 1)
    ke_main = jax.lax.shift_right_logical(ke, shift)
    acc_e = jnp.zeros((1, npad * expand), jnp.int32)
    for b in range(n // c):
        cc = cums[b * c:(b + 1) * c, :].astype(jnp.int32)
        acc_e = acc_e + jnp.sum((cc <= ke_main).astype(jnp.int32), axis=0,
                                keepdims=True)
    sub = ke[0:1, :] & (expand - 1)
    return acc, acc_e * expand + sub


def _shot_body(sent_ref, w1_ref, b1_ref, ln1s_ref, ln1b_ref, ln2s_ref,
               ln2b_ref, w2_ref, b2_ref, g1_ref, o_ref, h1_ref):
    j = pl.program_id(0)
    xn = _ln(sent_ref[...], ln1s_ref[...], ln1b_ref[...])
    h = jax.lax.dot_general(xn, w1_ref[...], (((1,), (1,)), ((), ())),
                            preferred_element_type=jnp.float32)
    h1_ref[j] = h + b1_ref[0, 0, :][None, :]

    @pl.when(j == pl.num_programs(0) - 1)
    def _():
        h1 = jnp.concatenate([h1_ref[b] for b in range(D // NBLK)], axis=1)
        h2 = _ln(h1, ln2s_ref[...], ln2b_ref[...])
        g = _gelu_exact(h2)
        s = jax.lax.dot_general(g, w2_ref[...], (((1,), (1,)), ((), ())),
                                preferred_element_type=jnp.float32)
        s = s + b2_ref[...] + g1_ref[...]
        p = _softmax1(s)
        p_row = jnp.swapaxes(p, 0, 1)
        pos = _topk_positions(p, p_row, KSHOT, 128)
        o_ref[...] = jnp.broadcast_to(pos, (8, 128))


def _shot_topk(sent, ln1s, ln1b, w1, b1, ln2s, ln2b, w2, b2, g1):
    nb = D // NBLK
    out = pl.pallas_call(
        _shot_body,
        grid=(nb,),
        in_specs=[
            pl.BlockSpec((NSHOT, D), lambda j: (0, 0)),
            pl.BlockSpec((NBLK, D), lambda j: (j, 0)),
            pl.BlockSpec((1, 1, NBLK), lambda j: (j, 0, 0)),
            pl.BlockSpec((D,), lambda j: (0,)),
            pl.BlockSpec((D,), lambda j: (0,)),
            pl.BlockSpec((D,), lambda j: (0,)),
            pl.BlockSpec((D,), lambda j: (0,)),
            pl.BlockSpec((2, D), lambda j: (0, 0)),
            pl.BlockSpec((1, 2), lambda j: (0, 0)),
            pl.BlockSpec((NSHOT, 2), lambda j: (0, 0)),
        ],
        out_specs=pl.BlockSpec((8, 128), lambda j: (0, 0)),
        out_shape=jax.ShapeDtypeStruct((8, 128), jnp.int32),
        scratch_shapes=[pltpu.VMEM((nb, NSHOT, NBLK), jnp.float32)],
    )(sent, w1, b1.reshape(nb, 1, NBLK), ln1s, ln1b, ln2s, ln2b, w2,
      b2.reshape(1, 2), g1)
    return out[0, :KSHOT]


def _gather_ln_body(idx_ref, x_ref, s_ref, b_ref, o_raw_ref, o_ln_ref):
    del idx_ref
    x = x_ref[...]
    o_raw_ref[...] = x
    o_ln_ref[...] = _ln(x, s_ref[...], b_ref[...])


def _shot_gather(input_embeds, shot_idx, ln1s, ln1b):
    tchunk = 128
    grid_spec = pltpu.PrefetchScalarGridSpec(
        num_scalar_prefetch=1,
        grid=(KSHOT, NTOK // tchunk),
        in_specs=[
            pl.BlockSpec((1, tchunk, D), lambda i, j, idx: (idx[i], j, 0)),
            pl.BlockSpec((D,), lambda i, j, idx: (0,)),
            pl.BlockSpec((D,), lambda i, j, idx: (0,)),
        ],
        out_specs=[
            pl.BlockSpec((1, tchunk, D), lambda i, j, idx: (i, j, 0)),
            pl.BlockSpec((1, tchunk, D), lambda i, j, idx: (i, j, 0)),
        ],
    )
    return pl.pallas_call(
        _gather_ln_body,
        grid_spec=grid_spec,
        out_shape=[jax.ShapeDtypeStruct((KSHOT, NTOK, D), jnp.float32),
                   jax.ShapeDtypeStruct((KSHOT, NTOK, D), jnp.float32)],
    )(shot_idx, input_embeds, ln1s, ln1b)


def _tokmm_body(xn_ref, w_ref, b_ref, o_ref):
    o_ref[...] = jax.lax.dot_general(
        xn_ref[...], w_ref[...], (((1,), (1,)), ((), ())),
        preferred_element_type=jnp.float32) + b_ref[0, 0, :][None, :]


K4NBLK = 256


def _token_matmul(xn, w1, b1):
    nb = D // K4NBLK
    return pl.pallas_call(
        _tokmm_body,
        grid=(nb,),
        in_specs=[
            pl.BlockSpec((FLAT, D), lambda j: (0, 0)),
            pl.BlockSpec((K4NBLK, D), lambda j: (j, 0)),
            pl.BlockSpec((1, 1, K4NBLK), lambda j: (j, 0, 0)),
        ],
        out_specs=pl.BlockSpec((FLAT, K4NBLK), lambda j: (0, j)),
        out_shape=jax.ShapeDtypeStruct((FLAT, D), jnp.float32),
    )(xn, w1, b1.reshape(nb, 1, K4NBLK))


def _tokscore_body(h_ref, ln2s_ref, ln2b_ref, w2_ref, b2_ref, g2_ref,
                   pcol_ref, prow_ref):
    h2 = _ln(h_ref[...], ln2s_ref[...], ln2b_ref[...])
    g = _gelu_exact(h2)
    s = jax.lax.dot_general(g, w2_ref[...], (((1,), (1,)), ((), ())),
                            preferred_element_type=jnp.float32)
    s = s + b2_ref[...] + g2_ref[...]
    p = _softmax1(s)
    pcol_ref[...] = p
    prow_ref[...] = jnp.swapaxes(p, 0, 1)


def _token_scores(h1, ln2s, ln2b, w2, b2, g2):
    nb = FLAT // SBLK
    return pl.pallas_call(
        _tokscore_body,
        grid=(nb,),
        in_specs=[
            pl.BlockSpec((SBLK, D), lambda i: (i, 0)),
            pl.BlockSpec((D,), lambda i: (0,)),
            pl.BlockSpec((D,), lambda i: (0,)),
            pl.BlockSpec((2, D), lambda i: (0, 0)),
            pl.BlockSpec((1, 2), lambda i: (0, 0)),
            pl.BlockSpec((SBLK, 2), lambda i: (i, 0)),
        ],
        out_specs=[pl.BlockSpec((SBLK, 1), lambda i: (i, 0)),
                   pl.BlockSpec((1, SBLK), lambda i: (0, i))],
        out_shape=[jax.ShapeDtypeStruct((FLAT, 1), jnp.float32),
                   jax.ShapeDtypeStruct((1, FLAT), jnp.float32)],
    )(h1, ln2s, ln2b, w2, b2.reshape(1, 2), g2)


def _toktopk_body(pcol_ref, prow_ref, o_pos_ref, o_exp_ref):
    pos, exp = _topk_positions(pcol_ref[...], prow_ref[...], KTOK, KTOK,
                               expand=RSUB)
    o_pos_ref[...] = pos
    o_exp_ref[...] = exp


def _token_topk(p_col, p_row):
    pos, exp = pl.pallas_call(
        _toktopk_body,
        grid=(1,),
        in_specs=[pl.BlockSpec((FLAT, 1), lambda i: (0, 0)),
                  pl.BlockSpec((1, FLAT), lambda i: (0, 0))],
        out_specs=[pl.BlockSpec((1, KTOK), lambda i: (0, 0)),
                   pl.BlockSpec((1, KTOK * RSUB), lambda i: (0, 0))],
        out_shape=[jax.ShapeDtypeStruct((1, KTOK), jnp.int32),
                   jax.ShapeDtypeStruct((1, KTOK * RSUB), jnp.int32)],
    )(p_col, p_row)
    return pos.reshape(KTOK), exp


RSUB = 16
DSUB = D // RSUB
GATHER_WIN = 128


def _final_gather(flat, exp_idx):
    xs = flat.reshape(FLAT * RSUB, DSUB)
    nidx = KTOK * RSUB
    mesh = plsc.VectorSubcoreMesh(core_axis_name="core",
                                  subcore_axis_name="subcore")

    @pl.kernel(out_type=jax.ShapeDtypeStruct((nidx, DSUB), jnp.float32),
               mesh=mesh)
    def gather_kernel(x_hbm, i_hbm, o_hbm):
        def body(i_vmem, o_vmem):
            pltpu.sync_copy(x_hbm.at[i_vmem.at[0]], o_vmem)

        pltpu.emit_pipeline(
            body,
            grid=(nidx // GATHER_WIN,),
            in_specs=[pl.BlockSpec((1, GATHER_WIN), lambda i: (0, i))],
            out_specs=[pl.BlockSpec((GATHER_WIN, DSUB), lambda i: (i, 0))],
            core_axis_name="subcore",
            dimension_semantics=(pltpu.PARALLEL,),
        )(i_hbm, o_hbm)

    return gather_kernel(xs, exp_idx).reshape(KTOK, D)


def kernel(input_embeds, sp_ln1_s, sp_ln1_b, sp_w1, sp_b1, sp_ln2_s, sp_ln2_b,
           sp_w2, sp_b2, tp_ln1_s, tp_ln1_b, tp_w1, tp_b1, tp_ln2_s, tp_ln2_b,
           tp_w2, tp_b2):
    gkey = jax.random.key(42)
    g1 = jax.random.gumbel(jax.random.fold_in(gkey, 0), (NSHOT, 2),
                           jnp.float32)
    g2 = jax.random.gumbel(jax.random.fold_in(gkey, 1), (FLAT, 2),
                           jnp.float32)

    sent = _shot_means(input_embeds)
    top_shot_positions = _shot_topk(sent, sp_ln1_s, sp_ln1_b, sp_w1, sp_b1,
                                    sp_ln2_s, sp_ln2_b, sp_w2, sp_b2, g1)
    pruned, xn = _shot_gather(input_embeds, top_shot_positions, tp_ln1_s,
                              tp_ln1_b)
    h1 = _token_matmul(xn.reshape(FLAT, D), tp_w1, tp_b1)
    p_col, p_row = _token_scores(h1, tp_ln2_s, tp_ln2_b, tp_w2, tp_b2, g2)
    top_token_positions, exp_idx = _token_topk(p_col, p_row)
    pruned_final = _final_gather(pruned.reshape(FLAT, D), exp_idx)
    return (pruned, pruned_final, top_shot_positions, top_token_positions)


# speedup vs baseline: 1.0570x; 1.0570x over previous
import jax
import jax.numpy as jnp
from jax.experimental import pallas as pl
from jax.experimental.pallas import tpu as pltpu
from jax.experimental.pallas import tpu_sc as plsc

D = 4096
NSHOT = 32
NTOK = 512
KSHOT = 4
KTOK = 1024
FLAT = KSHOT * NTOK

NBLK = 512
MBLK = 1024
SBLK = 256


def _ln(x, scale, bias, eps=1e-5):
    m = jnp.mean(x, axis=-1, keepdims=True)
    v = jnp.var(x, axis=-1, keepdims=True)
    return (x - m) / jnp.sqrt(v + eps) * scale + bias


def _gelu_exact(x):
    return 0.5 * x * (1.0 + jax.lax.erf(x * 0.7071067811865476))


def _softmax1(s):
    m = jnp.max(s, axis=-1, keepdims=True)
    e = jnp.exp(s - m)
    return e[:, 1:2] / (e[:, 0:1] + e[:, 1:2])


def _topk_positions(p_col, p_row, k, npad, expand=1):
    n = p_col.shape[0]
    c = min(SBLK, n)
    jrow = jax.lax.broadcasted_iota(jnp.int32, (c, n), 1)
    ranks = []
    for b in range(n // c):
        pc = p_col[b * c:(b + 1) * c, :]
        icol = jax.lax.broadcasted_iota(jnp.int32, (c, 1), 0) + b * c
        cond = (p_row > pc) | ((p_row == pc) & (jrow < icol))
        ranks.append(jnp.sum(cond.astype(jnp.float32), axis=1, keepdims=True))
    rank = jnp.concatenate(ranks, axis=0) if len(ranks) > 1 else ranks[0]
    sel = (rank < float(k)).astype(jnp.float32)
    sel_row = jnp.swapaxes(sel, 0, 1)
    cums = []
    for b in range(n // c):
        irow = jax.lax.broadcasted_iota(jnp.int32, (c, 1), 0) + b * c
        tri = (jrow <= irow).astype(jnp.float32)
        cums.append(jnp.sum(tri * sel_row, axis=1, keepdims=True))
    cums = jnp.concatenate(cums, axis=0) if len(cums) > 1 else cums[0]
    krow = jax.lax.broadcasted_iota(jnp.int32, (c, npad), 1)
    acc = jnp.zeros((1, npad), jnp.int32)
    for b in range(n // c):
        cc = cums[b * c:(b + 1) * c, :].astype(jnp.int32)
        acc = acc + jnp.sum((cc <= krow).astype(jnp.int32), axis=0,
                            keepdims=True)
    if expand == 1:
        return acc
    shift = expand.bit_length() - 1
    echunk = 4096
    ne = npad * expand
    parts = []
    for l in range(ne // echunk):
        ke = jax.lax.broadcasted_iota(jnp.int32, (c, echunk), 1) + l * echunk
        ke_main = jax.lax.shift_right_logical(ke, shift)
        acc_l = jnp.zeros((1, echunk), jnp.int32)
        for b in range(n // c):
            cc = cums[b * c:(b + 1) * c, :].astype(jnp.int32)
            acc_l = acc_l + jnp.sum((cc <= ke_main).astype(jnp.int32),
                                    axis=0, keepdims=True)
        sub_l = ke[0:1, :] & (expand - 1)
        parts.append(acc_l * expand + sub_l)
    exp = jnp.concatenate(parts, axis=1) if len(parts) > 1 else parts[0]
    return acc, exp


def _shot_body(x_ref, w1_ref, b1_ref, ln1s_ref, ln1b_ref, ln2s_ref,
               ln2b_ref, w2_ref, b2_ref, g1_ref, o_ref, sent_ref, h1_ref):
    t = pl.program_id(0)

    @pl.when(t < NSHOT)
    def _():
        sent_ref[pl.ds(t, 1), :] = jnp.mean(x_ref[...], axis=1)

    @pl.when(t >= NSHOT)
    def _():
        j = t - NSHOT
        sent = sent_ref[...]
        xn = _ln(sent, ln1s_ref[...], ln1b_ref[...])
        h = jax.lax.dot_general(xn, w1_ref[...], (((1,), (1,)), ((), ())),
                                preferred_element_type=jnp.float32)
        h1_ref[j] = h + b1_ref[0, 0, :][None, :]

    @pl.when(t == pl.num_programs(0) - 1)
    def _():
        h1 = jnp.concatenate([h1_ref[b] for b in range(D // NBLK)], axis=1)
        h2 = _ln(h1, ln2s_ref[...], ln2b_ref[...])
        g = _gelu_exact(h2)
        s = jax.lax.dot_general(g, w2_ref[...], (((1,), (1,)), ((), ())),
                                preferred_element_type=jnp.float32)
        s = s + b2_ref[...] + g1_ref[...]
        p = _softmax1(s)
        p_row = jnp.swapaxes(p, 0, 1)
        pos = _topk_positions(p, p_row, KSHOT, 128)
        o_ref[...] = jnp.broadcast_to(pos, (8, 128))


def _shot_topk(input_embeds, ln1s, ln1b, w1, b1, ln2s, ln2b, w2, b2, g1):
    nb = D // NBLK
    out = pl.pallas_call(
        _shot_body,
        grid=(NSHOT + nb,),
        in_specs=[
            pl.BlockSpec((1, NTOK, D),
                         lambda t: (jnp.minimum(t, NSHOT - 1), 0, 0)),
            pl.BlockSpec((NBLK, D),
                         lambda t: (jnp.maximum(t - NSHOT, 0), 0)),
            pl.BlockSpec((1, 1, NBLK),
                         lambda t: (jnp.maximum(t - NSHOT, 0), 0, 0)),
            pl.BlockSpec((D,), lambda t: (0,)),
            pl.BlockSpec((D,), lambda t: (0,)),
            pl.BlockSpec((D,), lambda t: (0,)),
            pl.BlockSpec((D,), lambda t: (0,)),
            pl.BlockSpec((2, D), lambda t: (0, 0)),
            pl.BlockSpec((1, 2), lambda t: (0, 0)),
            pl.BlockSpec((NSHOT, 2), lambda t: (0, 0)),
        ],
        out_specs=pl.BlockSpec((8, 128), lambda t: (0, 0)),
        out_shape=jax.ShapeDtypeStruct((8, 128), jnp.int32),
        scratch_shapes=[pltpu.VMEM((NSHOT, D), jnp.float32),
                        pltpu.VMEM((nb, NSHOT, NBLK), jnp.float32)],
    )(input_embeds, w1, b1.reshape(nb, 1, NBLK), ln1s, ln1b, ln2s, ln2b, w2,
      b2.reshape(1, 2), g1)
    return out[0, :KSHOT]


def _gather_ln_body(idx_ref, x_ref, s_ref, b_ref, o_raw_ref, o_ln_ref):
    del idx_ref
    x = x_ref[...]
    o_raw_ref[...] = x
    o_ln_ref[...] = _ln(x, s_ref[...], b_ref[...])


def _shot_gather(input_embeds, shot_idx, ln1s, ln1b):
    tchunk = 128
    grid_spec = pltpu.PrefetchScalarGridSpec(
        num_scalar_prefetch=1,
        grid=(KSHOT, NTOK // tchunk),
        in_specs=[
            pl.BlockSpec((1, tchunk, D), lambda i, j, idx: (idx[i], j, 0)),
            pl.BlockSpec((D,), lambda i, j, idx: (0,)),
            pl.BlockSpec((D,), lambda i, j, idx: (0,)),
        ],
        out_specs=[
            pl.BlockSpec((1, tchunk, D), lambda i, j, idx: (i, j, 0)),
            pl.BlockSpec((1, tchunk, D), lambda i, j, idx: (i, j, 0)),
        ],
    )
    return pl.pallas_call(
        _gather_ln_body,
        grid_spec=grid_spec,
        out_shape=[jax.ShapeDtypeStruct((KSHOT, NTOK, D), jnp.float32),
                   jax.ShapeDtypeStruct((KSHOT, NTOK, D), jnp.float32)],
    )(shot_idx, input_embeds, ln1s, ln1b)


def _tokmm_body(xn_ref, w_ref, b_ref, o_ref):
    o_ref[...] = jax.lax.dot_general(
        xn_ref[...], w_ref[...], (((1,), (1,)), ((), ())),
        preferred_element_type=jnp.float32) + b_ref[0, 0, :][None, :]


K4NBLK = 256


def _token_matmul(xn, w1, b1):
    nb = D // K4NBLK
    return pl.pallas_call(
        _tokmm_body,
        grid=(nb,),
        in_specs=[
            pl.BlockSpec((FLAT, D), lambda j: (0, 0)),
            pl.BlockSpec((K4NBLK, D), lambda j: (j, 0)),
            pl.BlockSpec((1, 1, K4NBLK), lambda j: (j, 0, 0)),
        ],
        out_specs=pl.BlockSpec((FLAT, K4NBLK), lambda j: (0, j)),
        out_shape=jax.ShapeDtypeStruct((FLAT, D), jnp.float32),
    )(xn, w1, b1.reshape(nb, 1, K4NBLK))


def _tokscore_body(h_ref, ln2s_ref, ln2b_ref, w2_ref, b2_ref, g2_ref,
                   pos_ref, exp_ref, pc_ref, pr_ref):
    i = pl.program_id(0)
    h2 = _ln(h_ref[...], ln2s_ref[...], ln2b_ref[...])
    g = _gelu_exact(h2)
    s = jax.lax.dot_general(g, w2_ref[...], (((1,), (1,)), ((), ())),
                            preferred_element_type=jnp.float32)
    s = s + b2_ref[...] + g2_ref[...]
    p = _softmax1(s)
    pc_ref[i] = p
    pr_ref[i] = jnp.swapaxes(p, 0, 1)

    @pl.when(i == pl.num_programs(0) - 1)
    def _():
        nb = FLAT // SBLK
        p_col = jnp.concatenate([pc_ref[b] for b in range(nb)], axis=0)
        p_row = jnp.concatenate([pr_ref[b] for b in range(nb)], axis=1)
        pos, exp = _topk_positions(p_col, p_row, KTOK, KTOK, expand=RSUB)
        pos_ref[...] = pos
        exp_ref[...] = exp


def _token_scores_topk(h1, ln2s, ln2b, w2, b2, g2):
    nb = FLAT // SBLK
    pos, exp = pl.pallas_call(
        _tokscore_body,
        grid=(nb,),
        in_specs=[
            pl.BlockSpec((SBLK, D), lambda i: (i, 0)),
            pl.BlockSpec((D,), lambda i: (0,)),
            pl.BlockSpec((D,), lambda i: (0,)),
            pl.BlockSpec((2, D), lambda i: (0, 0)),
            pl.BlockSpec((1, 2), lambda i: (0, 0)),
            pl.BlockSpec((SBLK, 2), lambda i: (i, 0)),
        ],
        out_specs=[pl.BlockSpec((1, KTOK), lambda i: (0, 0)),
                   pl.BlockSpec((1, KTOK * RSUB), lambda i: (0, 0))],
        out_shape=[jax.ShapeDtypeStruct((1, KTOK), jnp.int32),
                   jax.ShapeDtypeStruct((1, KTOK * RSUB), jnp.int32)],
        scratch_shapes=[pltpu.VMEM((nb, SBLK, 1), jnp.float32),
                        pltpu.VMEM((nb, 1, SBLK), jnp.float32)],
    )(h1, ln2s, ln2b, w2, b2.reshape(1, 2), g2)
    return pos.reshape(KTOK), exp


RSUB = 16
DSUB = D // RSUB
GATHER_WIN = 128


def _final_gather(flat, exp_idx):
    xs = flat.reshape(FLAT * RSUB, DSUB)
    nidx = KTOK * RSUB
    mesh = plsc.VectorSubcoreMesh(core_axis_name="core",
                                  subcore_axis_name="subcore")

    @pl.kernel(out_type=jax.ShapeDtypeStruct((nidx, DSUB), jnp.float32),
               mesh=mesh)
    def gather_kernel(x_hbm, i_hbm, o_hbm):
        def body(i_vmem, o_vmem):
            pltpu.sync_copy(x_hbm.at[i_vmem.at[0]], o_vmem)

        nblk = nidx // GATHER_WIN
        half = nblk // 2
        pltpu.emit_pipeline(
            body,
            grid=(2, half),
            in_specs=[pl.BlockSpec((1, GATHER_WIN),
                                   lambda c, i: (0, c * half + i))],
            out_specs=[pl.BlockSpec((GATHER_WIN, DSUB),
                                    lambda c, i: (c * half + i, 0))],
            core_axis_name=("core", "subcore"),
            dimension_semantics=(pltpu.PARALLEL, pltpu.PARALLEL),
        )(i_hbm, o_hbm)

    return gather_kernel(xs, exp_idx).reshape(KTOK, D)


def kernel(input_embeds, sp_ln1_s, sp_ln1_b, sp_w1, sp_b1, sp_ln2_s, sp_ln2_b,
           sp_w2, sp_b2, tp_ln1_s, tp_ln1_b, tp_w1, tp_b1, tp_ln2_s, tp_ln2_b,
           tp_w2, tp_b2):
    gkey = jax.random.key(42)
    g1 = jax.random.gumbel(jax.random.fold_in(gkey, 0), (NSHOT, 2),
                           jnp.float32)
    g2 = jax.random.gumbel(jax.random.fold_in(gkey, 1), (FLAT, 2),
                           jnp.float32)

    top_shot_positions = _shot_topk(input_embeds, sp_ln1_s, sp_ln1_b, sp_w1,
                                    sp_b1, sp_ln2_s, sp_ln2_b, sp_w2, sp_b2,
                                    g1)
    pruned, xn = _shot_gather(input_embeds, top_shot_positions, tp_ln1_s,
                              tp_ln1_b)
    h1 = _token_matmul(xn.reshape(FLAT, D), tp_w1, tp_b1)
    top_token_positions, exp_idx = _token_scores_topk(
        h1, tp_ln2_s, tp_ln2_b, tp_w2, tp_b2, g2)
    pruned_final = _final_gather(pruned.reshape(FLAT, D), exp_idx)
    return (pruned, pruned_final, top_shot_positions, top_token_positions)
